# baseline (speedup 1.0000x reference)
import functools

import jax
import jax.numpy as jnp
from jax import lax
from jax.experimental import pallas as pl
from jax.experimental.pallas import tpu as pltpu
from jax.experimental.pallas import tpu_sc as plsc

N_USERS = 25000
N_ITEMS = 25000
N_NODES = 50000
EMB = 64
N_LAYERS = 3

NC = 2
NS = 16
NW = NC * NS
CH = 128
EP = 802816
EPT32 = EP // NW
PNCH = EPT32 // CH
CAPB = PNCH + 4
CAPB2 = 2 * PNCH + 8

HALF0 = 25088
BROWS = HALF0 // NS
NPAD = 2 * HALF0
BMAGIC = 2675

_GATHER_DNUMS = lax.GatherDimensionNumbers(
    offset_dims=(), collapsed_slice_dims=(0,), start_index_map=(0,))


def _lane_splat(v16, e):
    idx = jnp.full((16, 1), e, jnp.int32)
    return lax.gather(v16, idx, _GATHER_DNUMS, (1,),
                      mode=lax.GatherScatterMode.PROMISE_IN_BOUNDS)


def _lane_extract(v16, lane, iota):
    return jnp.max(jnp.where(iota == lane, v16, 0))


def _part_body(e2_hbm, w2_hbm, eo_hbm, wo_hbm, cnt_hbm,
               ebuf, wbuf, cbuf, srA, dsA, wrA, srB, dsB, wrB):
    c = lax.axis_index("c")
    s = lax.axis_index("s")
    wid = c * NS + s
    iota = lax.iota(jnp.int32, 16)
    zi = jnp.zeros((16,), jnp.int32)
    zf = jnp.zeros((16,), jnp.float32)

    for g in range(16):
        srA[pl.ds(g * 16, 16)] = zi
        dsA[pl.ds(g * 16, 16)] = zi
        srB[pl.ds(g * 16, 16)] = zi
        dsB[pl.ds(g * 16, 16)] = zi
        wrA[pl.ds(g * 16, 16)] = zf
        wrB[pl.ds(g * 16, 16)] = zf

    rings = ((srA, dsA, wrA), (srB, dsB, wrB))

    def flush(h, cur):
        sr, ds_, wr = rings[h]
        rb = (cur >> 7) & 1
        blk = cur >> 7
        pltpu.sync_copy(sr.at[pl.ds(rb * 128, 128)],
                        eo_hbm.at[h, wid, blk, 0])
        pltpu.sync_copy(ds_.at[pl.ds(rb * 128, 128)],
                        eo_hbm.at[h, wid, blk, 1])
        pltpu.sync_copy(wr.at[pl.ds(rb * 128, 128)],
                        wo_hbm.at[h, wid, blk])

    def chunk(i, curs):
        curA, curB = curs
        startA, startB = curA, curB
        row = wid * PNCH + i
        pltpu.sync_copy(e2_hbm.at[row], ebuf)
        pltpu.sync_copy(w2_hbm.at[row], wbuf)
        for g in range(CH // 16):
            o = g * 16
            src16 = ebuf[0, pl.ds(o, 16)]
            dst16 = ebuf[1, pl.ds(o, 16)]
            w16 = wbuf[pl.ds(o, 16)]
            mA = dst16 < HALF0
            dloc16 = jnp.where(mA, dst16, dst16 - HALF0)
            for h, m, cur in ((0, mA, curA), (1, ~mA, curB)):
                sr, ds_, wr = rings[h]
                cm = plsc.cumsum(jnp.where(m, 1, 0))
                pos = (cur + cm - 1) & 255
                plsc.store_scatter(sr, [pos], src16, mask=m)
                plsc.store_scatter(ds_, [pos], dloc16, mask=m)
                plsc.store_scatter(wr, [pos], w16, mask=m)
                if h == 0:
                    curA = cur + jnp.max(cm)
                else:
                    curB = cur + jnp.max(cm)
        for h, st, ncur in ((0, startA, curA), (1, startB, curB)):
            @pl.when((ncur >> 7) != (st >> 7))
            def _(h=h, st=st):
                flush(h, st)
        return (curA, curB)

    curA, curB = lax.fori_loop(0, PNCH, chunk, (jnp.int32(0), jnp.int32(0)))

    nblks = []
    for h, cur in ((0, curA), (1, curB)):
        sr, ds_, wr = rings[h]
        rem = cur & 127
        for g in range(CH // 16):
            pos = (cur + g * 16 + iota) & 255
            mm = (g * 16 + iota) < (128 - rem)
            plsc.store_scatter(wr, [pos], jnp.zeros((16,), jnp.float32),
                               mask=mm)
        flush(h, cur)
        nblks.append((cur >> 7) + 1)

    cv = (jnp.where(iota == 0, nblks[0], 0)
          + jnp.where(iota == 1, nblks[1], 0))
    cbuf[pl.ds(0, 16)] = cv
    pltpu.sync_copy(cbuf, cnt_hbm.at[wid])


@functools.cache
def _partition_fn():
  return pl.kernel(
    _part_body,
    out_type=(
        jax.ShapeDtypeStruct((2, NW, CAPB, 2, CH), jnp.int32),
        jax.ShapeDtypeStruct((2, NW, CAPB, CH), jnp.float32),
        jax.ShapeDtypeStruct((NW, 16), jnp.int32),
    ),
    mesh=plsc.VectorSubcoreMesh(core_axis_name="c", subcore_axis_name="s",
                                num_cores=NC, num_subcores=NS),
    compiler_params=pltpu.CompilerParams(use_tc_tiling_on_sc=False,
                                         needs_layout_passes=False),
    scratch_types=[
        pltpu.VMEM((2, CH), jnp.int32),
        pltpu.VMEM((CH,), jnp.float32),
        pltpu.VMEM((16,), jnp.int32),
        pltpu.VMEM((256,), jnp.int32),
        pltpu.VMEM((256,), jnp.int32),
        pltpu.VMEM((256,), jnp.float32),
        pltpu.VMEM((256,), jnp.int32),
        pltpu.VMEM((256,), jnp.int32),
        pltpu.VMEM((256,), jnp.float32),
    ],
  )


def _part2_body(eo1_hbm, wo1_hbm, cnt1_hbm, eo2_hbm, wo2_hbm, cnt2_hbm,
                ebuf, wbuf, cbuf, cntv, sr2, ds2, wr2, sf):
    c = lax.axis_index("c")
    s = lax.axis_index("s")
    p = c * NS + s
    iota = lax.iota(jnp.int32, 16)
    zi = jnp.zeros((16,), jnp.int32)
    zf = jnp.zeros((16,), jnp.float32)

    def zring(i, carry):
        for b in range(NS):
            sr2[b, pl.ds(i * 16, 16)] = zi
            ds2[b, pl.ds(i * 16, 16)] = zi
            wr2[b, pl.ds(i * 16, 16)] = zf
        return carry
    lax.fori_loop(0, 16, zring, 0)

    t1 = s
    t2 = NS + s
    pltpu.sync_copy(cnt1_hbm.at[t1], cntv.at[0])
    pltpu.sync_copy(cnt1_hbm.at[t2], cntv.at[1])
    n1 = _lane_extract(cntv[0, pl.ds(0, 16)], c, iota)
    n2 = _lane_extract(cntv[1, pl.ds(0, 16)], c, iota)
    nbt = n1 + n2

    def flush(b, cur):
        rb = (cur >> 7) & 1
        blk = cur >> 7
        pltpu.async_copy(sr2.at[b, pl.ds(rb * 128, 128)],
                         eo2_hbm.at[p, b, blk, 0], sf)
        pltpu.async_copy(ds2.at[b, pl.ds(rb * 128, 128)],
                         eo2_hbm.at[p, b, blk, 1], sf)
        pltpu.async_copy(wr2.at[b, pl.ds(rb * 128, 128)],
                         wo2_hbm.at[p, b, blk], sf)

    def chunk(i, carry):
        curs = list(carry[:NS])
        nf = carry[NS]
        t = jnp.where(i < n1, t1, t2)
        blk = jnp.where(i < n1, i, i - n1)
        pltpu.sync_copy(eo1_hbm.at[c, t, blk], ebuf)
        pltpu.sync_copy(wo1_hbm.at[c, t, blk], wbuf)
        starts = list(curs)
        for g in range(CH // 16):
            o = g * 16
            src16 = ebuf[0, pl.ds(o, 16)]
            dl16 = ebuf[1, pl.ds(o, 16)]
            w16 = wbuf[pl.ds(o, 16)]
            bkt16 = (dl16 * BMAGIC) >> 22
            dloc16 = dl16 - bkt16 * BROWS
            for b in range(NS):
                m = bkt16 == b
                cm = plsc.cumsum(jnp.where(m, 1, 0))
                cur = curs[b]
                pos = (cur + cm - 1) & 255
                plsc.store_scatter(sr2.at[b], [pos], src16, mask=m)
                plsc.store_scatter(ds2.at[b], [pos], dloc16, mask=m)
                plsc.store_scatter(wr2.at[b], [pos], w16, mask=m)
                curs[b] = cur + jnp.max(cm)
        for b in range(NS):
            crossed = (curs[b] >> 7) != (starts[b] >> 7)

            @pl.when(crossed)
            def _(b=b, st=starts[b]):
                flush(b, st)
            nf = nf + jnp.where(crossed, 3, 0)
        return tuple(curs) + (nf,)

    init = tuple(jnp.int32(0) for _ in range(NS)) + (jnp.int32(0),)
    fin = lax.fori_loop(0, nbt, chunk, init)
    curs = fin[:NS]
    nf = fin[NS]

    cv = jnp.zeros((16,), jnp.int32)
    for b in range(NS):
        cur = curs[b]
        rem = cur & 127
        for g in range(CH // 16):
            pos = (cur + g * 16 + iota) & 255
            mm = (g * 16 + iota) < (128 - rem)
            plsc.store_scatter(wr2.at[b], [pos], zf, mask=mm)
        flush(b, cur)
        nf = nf + 3
        cv = cv + jnp.where(iota == b, (cur >> 7) + 1, 0)

    def drain(i, carry):
        pltpu.make_async_copy(sr2.at[0, pl.ds(0, 128)],
                              eo2_hbm.at[0, 0, 0, 0], sf).wait()
        return carry
    lax.fori_loop(0, nf, drain, 0)

    cbuf[pl.ds(0, 16)] = cv
    pltpu.sync_copy(cbuf, cnt2_hbm.at[p])


@functools.cache
def _partition2_fn():
  return pl.kernel(
    _part2_body,
    out_type=(
        jax.ShapeDtypeStruct((NW, NS, CAPB2, 2, CH), jnp.int32),
        jax.ShapeDtypeStruct((NW, NS, CAPB2, CH), jnp.float32),
        jax.ShapeDtypeStruct((NW, 16), jnp.int32),
    ),
    mesh=plsc.VectorSubcoreMesh(core_axis_name="c", subcore_axis_name="s",
                                num_cores=NC, num_subcores=NS),
    compiler_params=pltpu.CompilerParams(use_tc_tiling_on_sc=False,
                                         needs_layout_passes=False),
    scratch_types=[
        pltpu.VMEM((2, CH), jnp.int32),
        pltpu.VMEM((CH,), jnp.float32),
        pltpu.VMEM((16,), jnp.int32),
        pltpu.VMEM((2, 16), jnp.int32),
        pltpu.VMEM((NS, 256), jnp.int32),
        pltpu.VMEM((NS, 256), jnp.int32),
        pltpu.VMEM((NS, 256), jnp.float32),
        pltpu.SemaphoreType.DMA,
    ],
  )


def _layer_body(x_hbm, eo_hbm, wo_hbm, cnt_hbm, out_hbm,
                acc, ebuf, wbuf, cntv, rows3,
                sg0, sg1, sg2, se0, se1, se2):
    c = lax.axis_index("c")
    s = lax.axis_index("s")
    sg = (sg0, sg1, sg2)
    se = (se0, se1, se2)
    iota = lax.iota(jnp.int32, 16)
    cols = [(k * 16 + iota) for k in range(4)]

    for r in range(NS):
        pltpu.sync_copy(cnt_hbm.at[c * NS + r], cntv.at[r])
    cnts = jnp.zeros((16,), jnp.int32)
    for r in range(NS):
        nr = _lane_extract(cntv[r, pl.ds(0, 16)], s, iota)
        cnts = cnts + jnp.where(iota == r, nr, 0)
    prefix = plsc.cumsum(cnts)
    excl = prefix - cnts
    nbt = jnp.max(prefix)
    nbt3 = ((nbt + 2) // 3) * 3

    zf = jnp.zeros((16,), jnp.float32)

    def zacc(i, carry):
        for k in range(4):
            acc[i, pl.ds(k * 16, 16)] = zf
        return carry
    lax.fori_loop(0, BROWS, zacc, 0)

    def fire_edata(i, q):
        iv = jnp.minimum(i, nbt - 1)
        mle = prefix <= iv
        r = jnp.max(plsc.all_reduce_population_count(mle))
        blk = iv - _lane_extract(excl, r, iota)
        pltpu.async_copy(eo_hbm.at[c * NS + r, s, blk], ebuf.at[q], se[q])
        pltpu.async_copy(wo_hbm.at[c * NS + r, s, blk], wbuf.at[q], se[q])

    def wait_edata(q):
        pltpu.make_async_copy(eo_hbm.at[0, 0, 0], ebuf.at[q], se[q]).wait()
        pltpu.make_async_copy(wo_hbm.at[0, 0, 0], wbuf.at[q], se[q]).wait()

    def fire_gather(q):
        pltpu.async_copy(x_hbm.at[ebuf.at[q, 0]], rows3.at[q], sg[q])

    def wait_gather(q):
        pltpu.make_async_copy(x_hbm.at[ebuf.at[q, 0]], rows3.at[q],
                              sg[q]).wait()

    fire_edata(jnp.int32(0), 0)
    fire_edata(jnp.int32(1), 1)
    wait_edata(0)
    fire_gather(0)
    fire_edata(jnp.int32(2), 2)
    wait_edata(1)
    fire_gather(1)

    def iter3(i3, carry):
        for k in range(3):
            ci = i3 * 3 + k
            q = k
            wait_edata((k + 2) % 3)
            fire_gather((k + 2) % 3)
            wait_gather(q)
            valid = ci < nbt

            def blk_fn(bi, carry2, _q=q, _valid=valid):
                jb = bi * 16
                w16 = jnp.where(_valid, wbuf[_q, pl.ds(jb, 16)], 0.0)
                dl16 = ebuf[_q, 1, pl.ds(jb, 16)]
                for e in range(0, 16, 2):
                    j0 = jb + e
                    j1 = jb + e + 1
                    ws0 = _lane_splat(w16, e)
                    ws1 = _lane_splat(w16, e + 1)
                    dr0 = _lane_splat(dl16, e)
                    dr1 = _lane_splat(dl16, e + 1)
                    vals = [rows3[_q, j0, pl.ds(kk * 16, 16)]
                            for kk in range(4)]
                    vals += [rows3[_q, j1, pl.ds(kk * 16, 16)]
                             for kk in range(4)]
                    for kk in range(4):
                        plsc.addupdate_scatter(acc, [dr0, cols[kk]],
                                               vals[kk] * ws0)
                    for kk in range(4):
                        plsc.addupdate_scatter(acc, [dr1, cols[kk]],
                                               vals[4 + kk] * ws1)
                return carry2
            lax.fori_loop(0, CH // 16, blk_fn, 0)
            fire_edata(ci + 3, q)
        return carry
    lax.fori_loop(0, nbt3 // 3, iter3, 0)

    wait_gather(0)
    wait_gather(1)
    wait_edata(2)

    g0 = (c * NS + s) * BROWS
    nfw, rw = divmod(BROWS, 512)
    for q in range(nfw):
        pltpu.sync_copy(acc.at[pl.ds(q * 512, 512)],
                        out_hbm.at[pl.ds(g0 + q * 512, 512)])
    if rw:
        pltpu.sync_copy(acc.at[pl.ds(nfw * 512, rw)],
                        out_hbm.at[pl.ds(g0 + nfw * 512, rw)])


@functools.cache
def _sc_layer_fn():
  return pl.kernel(
    _layer_body,
    out_type=jax.ShapeDtypeStruct((NPAD, EMB), jnp.float32),
    mesh=plsc.VectorSubcoreMesh(core_axis_name="c", subcore_axis_name="s",
                                num_cores=NC, num_subcores=NS),
    compiler_params=pltpu.CompilerParams(use_tc_tiling_on_sc=False,
                                         needs_layout_passes=False),
    scratch_types=[
        pltpu.VMEM((BROWS, EMB), jnp.float32),
        pltpu.VMEM((3, 2, CH), jnp.int32),
        pltpu.VMEM((3, CH), jnp.float32),
        pltpu.VMEM((NS, 16), jnp.int32),
        pltpu.VMEM((3, CH, EMB), jnp.float32),
        pltpu.SemaphoreType.DMA,
        pltpu.SemaphoreType.DMA,
        pltpu.SemaphoreType.DMA,
        pltpu.SemaphoreType.DMA,
        pltpu.SemaphoreType.DMA,
        pltpu.SemaphoreType.DMA,
    ],
  )


def _mean_body(a, b, c, d, o):
    o[...] = (a[...] + b[...] + c[...] + d[...]) * 0.25


_mean = pl.pallas_call(
    _mean_body,
    grid=(250,),
    in_specs=[pl.BlockSpec((200, EMB), lambda i: (i, 0))] * 4,
    out_specs=pl.BlockSpec((200, EMB), lambda i: (i, 0)),
    out_shape=jax.ShapeDtypeStruct((N_NODES, EMB), jnp.float32),
)


def kernel(user_weight, item_weight, edge_index, edge_weight):
    x0 = jnp.concatenate(
        [user_weight, item_weight,
         jnp.zeros((NPAD - N_NODES, EMB), jnp.float32)], axis=0)
    src = edge_index[1]
    dst = edge_index[0]
    pad = EP - src.shape[0]
    src_p = jnp.concatenate([src, jnp.zeros((pad,), jnp.int32)])
    dst_p = jnp.concatenate([dst, jnp.full((pad,), N_NODES, jnp.int32)])
    w_p = jnp.concatenate([edge_weight, jnp.zeros((pad,), jnp.float32)])
    e2 = jnp.stack([src_p.reshape(EP // CH, CH),
                    dst_p.reshape(EP // CH, CH)], axis=1)
    w2 = w_p.reshape(EP // CH, CH)

    eo1, wo1, cnt1 = _partition_fn()(e2, w2)
    eo2, wo2, cnt2 = _partition2_fn()(eo1, wo1, cnt1)

    sc_layer = _sc_layer_fn()
    xs = [x0]
    cur = x0
    for _ in range(N_LAYERS):
        cur = sc_layer(cur, eo2, wo2, cnt2)
        xs.append(cur)
    return _mean(*xs)


# speedup vs baseline: 1.1039x; 1.1039x over previous
import functools

import jax
import jax.numpy as jnp
from jax import lax
from jax.experimental import pallas as pl
from jax.experimental.pallas import tpu as pltpu
from jax.experimental.pallas import tpu_sc as plsc

N_USERS = 25000
N_ITEMS = 25000
N_NODES = 50000
EMB = 64
N_LAYERS = 3

NC = 2
NS = 16
NW = NC * NS
CH = 128
EP = 802816
EPT32 = EP // NW
PNCH = EPT32 // CH
CAPB = PNCH + 4
CAPB2 = 2 * PNCH + 8

HALF0 = 25088
BROWS = HALF0 // NS
NPAD = 2 * HALF0
BMAGIC = 2675

_GATHER_DNUMS = lax.GatherDimensionNumbers(
    offset_dims=(), collapsed_slice_dims=(0,), start_index_map=(0,))


def _lane_splat(v16, e):
    idx = jnp.full((16, 1), e, jnp.int32)
    return lax.gather(v16, idx, _GATHER_DNUMS, (1,),
                      mode=lax.GatherScatterMode.PROMISE_IN_BOUNDS)


def _lane_extract(v16, lane, iota):
    return jnp.max(jnp.where(iota == lane, v16, 0))


def _part_body(e2_hbm, w2_hbm, eo_hbm, wo_hbm, cnt_hbm,
               ebuf, wbuf, cbuf, srA, dsA, wrA, srB, dsB, wrB):
    c = lax.axis_index("c")
    s = lax.axis_index("s")
    wid = c * NS + s
    iota = lax.iota(jnp.int32, 16)
    zi = jnp.zeros((16,), jnp.int32)
    zf = jnp.zeros((16,), jnp.float32)

    for g in range(16):
        srA[pl.ds(g * 16, 16)] = zi
        dsA[pl.ds(g * 16, 16)] = zi
        srB[pl.ds(g * 16, 16)] = zi
        dsB[pl.ds(g * 16, 16)] = zi
        wrA[pl.ds(g * 16, 16)] = zf
        wrB[pl.ds(g * 16, 16)] = zf

    rings = ((srA, dsA, wrA), (srB, dsB, wrB))

    def flush(h, cur):
        sr, ds_, wr = rings[h]
        rb = (cur >> 7) & 1
        blk = cur >> 7
        pltpu.sync_copy(sr.at[pl.ds(rb * 128, 128)],
                        eo_hbm.at[h, wid, blk, 0])
        pltpu.sync_copy(ds_.at[pl.ds(rb * 128, 128)],
                        eo_hbm.at[h, wid, blk, 1])
        pltpu.sync_copy(wr.at[pl.ds(rb * 128, 128)],
                        wo_hbm.at[h, wid, blk])

    def chunk(i, curs):
        curA, curB = curs
        startA, startB = curA, curB
        row = wid * PNCH + i
        pltpu.sync_copy(e2_hbm.at[row], ebuf)
        pltpu.sync_copy(w2_hbm.at[row], wbuf)
        for g in range(CH // 16):
            o = g * 16
            src16 = ebuf[0, pl.ds(o, 16)]
            dst16 = ebuf[1, pl.ds(o, 16)]
            w16 = wbuf[pl.ds(o, 16)]
            mA = dst16 < HALF0
            dloc16 = jnp.where(mA, dst16, dst16 - HALF0)
            for h, m, cur in ((0, mA, curA), (1, ~mA, curB)):
                sr, ds_, wr = rings[h]
                cm = plsc.cumsum(jnp.where(m, 1, 0))
                pos = (cur + cm - 1) & 255
                plsc.store_scatter(sr, [pos], src16, mask=m)
                plsc.store_scatter(ds_, [pos], dloc16, mask=m)
                plsc.store_scatter(wr, [pos], w16, mask=m)
                if h == 0:
                    curA = cur + jnp.max(cm)
                else:
                    curB = cur + jnp.max(cm)
        for h, st, ncur in ((0, startA, curA), (1, startB, curB)):
            @pl.when((ncur >> 7) != (st >> 7))
            def _(h=h, st=st):
                flush(h, st)
        return (curA, curB)

    curA, curB = lax.fori_loop(0, PNCH, chunk, (jnp.int32(0), jnp.int32(0)))

    nblks = []
    for h, cur in ((0, curA), (1, curB)):
        sr, ds_, wr = rings[h]
        rem = cur & 127
        for g in range(CH // 16):
            pos = (cur + g * 16 + iota) & 255
            mm = (g * 16 + iota) < (128 - rem)
            plsc.store_scatter(wr, [pos], jnp.zeros((16,), jnp.float32),
                               mask=mm)
        flush(h, cur)
        nblks.append((cur >> 7) + 1)

    cv = (jnp.where(iota == 0, nblks[0], 0)
          + jnp.where(iota == 1, nblks[1], 0))
    cbuf[pl.ds(0, 16)] = cv
    pltpu.sync_copy(cbuf, cnt_hbm.at[wid])


@functools.cache
def _partition_fn():
  return pl.kernel(
    _part_body,
    out_type=(
        jax.ShapeDtypeStruct((2, NW, CAPB, 2, CH), jnp.int32),
        jax.ShapeDtypeStruct((2, NW, CAPB, CH), jnp.float32),
        jax.ShapeDtypeStruct((NW, 16), jnp.int32),
    ),
    mesh=plsc.VectorSubcoreMesh(core_axis_name="c", subcore_axis_name="s",
                                num_cores=NC, num_subcores=NS),
    compiler_params=pltpu.CompilerParams(use_tc_tiling_on_sc=False,
                                         needs_layout_passes=False),
    scratch_types=[
        pltpu.VMEM((2, CH), jnp.int32),
        pltpu.VMEM((CH,), jnp.float32),
        pltpu.VMEM((16,), jnp.int32),
        pltpu.VMEM((256,), jnp.int32),
        pltpu.VMEM((256,), jnp.int32),
        pltpu.VMEM((256,), jnp.float32),
        pltpu.VMEM((256,), jnp.int32),
        pltpu.VMEM((256,), jnp.int32),
        pltpu.VMEM((256,), jnp.float32),
    ],
  )


def _part2_body(eo1_hbm, wo1_hbm, cnt1_hbm, eo2_hbm, wo2_hbm, cnt2_hbm,
                ebuf, wbuf, cbuf, cntv, sr2, ds2, wr2, sf):
    c = lax.axis_index("c")
    s = lax.axis_index("s")
    p = c * NS + s
    iota = lax.iota(jnp.int32, 16)
    zi = jnp.zeros((16,), jnp.int32)
    zf = jnp.zeros((16,), jnp.float32)

    def zring(i, carry):
        for b in range(NS):
            sr2[b, pl.ds(i * 16, 16)] = zi
            ds2[b, pl.ds(i * 16, 16)] = zi
            wr2[b, pl.ds(i * 16, 16)] = zf
        return carry
    lax.fori_loop(0, 16, zring, 0)

    t1 = s
    t2 = NS + s
    pltpu.sync_copy(cnt1_hbm.at[t1], cntv.at[0])
    pltpu.sync_copy(cnt1_hbm.at[t2], cntv.at[1])
    n1 = _lane_extract(cntv[0, pl.ds(0, 16)], c, iota)
    n2 = _lane_extract(cntv[1, pl.ds(0, 16)], c, iota)
    nbt = n1 + n2

    def flush(b, cur):
        rb = (cur >> 7) & 1
        blk = cur >> 7
        pltpu.async_copy(sr2.at[b, pl.ds(rb * 128, 128)],
                         eo2_hbm.at[p, b, blk, 0], sf)
        pltpu.async_copy(ds2.at[b, pl.ds(rb * 128, 128)],
                         eo2_hbm.at[p, b, blk, 1], sf)
        pltpu.async_copy(wr2.at[b, pl.ds(rb * 128, 128)],
                         wo2_hbm.at[p, b, blk], sf)

    def chunk(i, carry):
        curs = list(carry[:NS])
        nf = carry[NS]
        t = jnp.where(i < n1, t1, t2)
        blk = jnp.where(i < n1, i, i - n1)
        pltpu.sync_copy(eo1_hbm.at[c, t, blk], ebuf)
        pltpu.sync_copy(wo1_hbm.at[c, t, blk], wbuf)
        starts = list(curs)
        for g in range(CH // 16):
            o = g * 16
            src16 = ebuf[0, pl.ds(o, 16)]
            dl16 = ebuf[1, pl.ds(o, 16)]
            w16 = wbuf[pl.ds(o, 16)]
            bkt16 = (dl16 * BMAGIC) >> 22
            dloc16 = dl16 - bkt16 * BROWS
            for b in range(NS):
                m = bkt16 == b
                cm = plsc.cumsum(jnp.where(m, 1, 0))
                cur = curs[b]
                pos = (cur + cm - 1) & 255
                plsc.store_scatter(sr2.at[b], [pos], src16, mask=m)
                plsc.store_scatter(ds2.at[b], [pos], dloc16, mask=m)
                plsc.store_scatter(wr2.at[b], [pos], w16, mask=m)
                curs[b] = cur + jnp.max(cm)
        for b in range(NS):
            crossed = (curs[b] >> 7) != (starts[b] >> 7)

            @pl.when(crossed)
            def _(b=b, st=starts[b]):
                flush(b, st)
            nf = nf + jnp.where(crossed, 3, 0)
        return tuple(curs) + (nf,)

    init = tuple(jnp.int32(0) for _ in range(NS)) + (jnp.int32(0),)
    fin = lax.fori_loop(0, nbt, chunk, init)
    curs = fin[:NS]
    nf = fin[NS]

    cv = jnp.zeros((16,), jnp.int32)
    for b in range(NS):
        cur = curs[b]
        rem = cur & 127
        for g in range(CH // 16):
            pos = (cur + g * 16 + iota) & 255
            mm = (g * 16 + iota) < (128 - rem)
            plsc.store_scatter(wr2.at[b], [pos], zf, mask=mm)
        flush(b, cur)
        nf = nf + 3
        cv = cv + jnp.where(iota == b, (cur >> 7) + 1, 0)

    def drain(i, carry):
        pltpu.make_async_copy(sr2.at[0, pl.ds(0, 128)],
                              eo2_hbm.at[0, 0, 0, 0], sf).wait()
        return carry
    lax.fori_loop(0, nf, drain, 0)

    cbuf[pl.ds(0, 16)] = cv
    pltpu.sync_copy(cbuf, cnt2_hbm.at[p])


@functools.cache
def _partition2_fn():
  return pl.kernel(
    _part2_body,
    out_type=(
        jax.ShapeDtypeStruct((NW, NS, CAPB2, 2, CH), jnp.int32),
        jax.ShapeDtypeStruct((NW, NS, CAPB2, CH), jnp.float32),
        jax.ShapeDtypeStruct((NW, 16), jnp.int32),
    ),
    mesh=plsc.VectorSubcoreMesh(core_axis_name="c", subcore_axis_name="s",
                                num_cores=NC, num_subcores=NS),
    compiler_params=pltpu.CompilerParams(use_tc_tiling_on_sc=False,
                                         needs_layout_passes=False),
    scratch_types=[
        pltpu.VMEM((2, CH), jnp.int32),
        pltpu.VMEM((CH,), jnp.float32),
        pltpu.VMEM((16,), jnp.int32),
        pltpu.VMEM((2, 16), jnp.int32),
        pltpu.VMEM((NS, 256), jnp.int32),
        pltpu.VMEM((NS, 256), jnp.int32),
        pltpu.VMEM((NS, 256), jnp.float32),
        pltpu.SemaphoreType.DMA,
    ],
  )


def _layer_body(x_hbm, eo_hbm, wo_hbm, cnt_hbm, out_hbm,
                acc, ebuf, wbuf, cntv, rows2,
                sg0, sg1, se0, se1, se2, se3):
    c = lax.axis_index("c")
    s = lax.axis_index("s")
    sg = (sg0, sg1)
    se = (se0, se1, se2, se3)
    iota = lax.iota(jnp.int32, 16)
    cols = [(k * 16 + iota) for k in range(4)]

    for r in range(NS):
        pltpu.sync_copy(cnt_hbm.at[c * NS + r], cntv.at[r])
    cnts = jnp.zeros((16,), jnp.int32)
    for r in range(NS):
        nr = _lane_extract(cntv[r, pl.ds(0, 16)], s, iota)
        cnts = cnts + jnp.where(iota == r, nr, 0)
    prefix = plsc.cumsum(cnts)
    excl = prefix - cnts
    nbt = jnp.max(prefix)
    nbt4 = ((nbt + 3) >> 2) << 2

    zf = jnp.zeros((16,), jnp.float32)

    def zacc(i, carry):
        for k in range(4):
            acc[i, pl.ds(k * 16, 16)] = zf
        return carry
    lax.fori_loop(0, BROWS, zacc, 0)

    def fire_edata(i, q):
        iv = jnp.minimum(i, nbt - 1)
        mle = prefix <= iv
        r = jnp.max(plsc.all_reduce_population_count(mle))
        blk = iv - _lane_extract(excl, r, iota)
        pltpu.async_copy(eo_hbm.at[c * NS + r, s, blk], ebuf.at[q], se[q])
        pltpu.async_copy(wo_hbm.at[c * NS + r, s, blk], wbuf.at[q], se[q])

    def wait_edata(q):
        pltpu.make_async_copy(eo_hbm.at[0, 0, 0], ebuf.at[q], se[q]).wait()
        pltpu.make_async_copy(wo_hbm.at[0, 0, 0], wbuf.at[q], se[q]).wait()

    def fire_gather(b, q):
        pltpu.async_copy(x_hbm.at[ebuf.at[q, 0]], rows2.at[b], sg[b])

    def wait_gather(b, q):
        pltpu.make_async_copy(x_hbm.at[ebuf.at[q, 0]], rows2.at[b],
                              sg[b]).wait()

    fire_edata(jnp.int32(0), 0)
    fire_edata(jnp.int32(1), 1)
    wait_edata(0)
    fire_gather(0, 0)
    fire_edata(jnp.int32(2), 2)

    def iter4(i4, carry):
        for k in range(4):
            ci = i4 * 4 + k
            q = k
            b = k & 1
            wait_edata((k + 1) % 4)
            fire_gather(1 - b, (k + 1) % 4)
            fire_edata(ci + 3, (k + 3) % 4)
            wait_gather(b, q)
            valid = ci < nbt

            def blk_fn(bi, carry2, _b=b, _q=q, _valid=valid):
                jb = bi * 16
                w16 = jnp.where(_valid, wbuf[_q, pl.ds(jb, 16)], 0.0)
                dl16 = ebuf[_q, 1, pl.ds(jb, 16)]
                for e in range(0, 16, 2):
                    j0 = jb + e
                    j1 = jb + e + 1
                    ws0 = _lane_splat(w16, e)
                    ws1 = _lane_splat(w16, e + 1)
                    dr0 = _lane_splat(dl16, e)
                    dr1 = _lane_splat(dl16, e + 1)
                    vals = [rows2[_b, j0, pl.ds(kk * 16, 16)]
                            for kk in range(4)]
                    vals += [rows2[_b, j1, pl.ds(kk * 16, 16)]
                             for kk in range(4)]
                    for kk in range(4):
                        plsc.addupdate_scatter(acc, [dr0, cols[kk]],
                                               vals[kk] * ws0)
                    for kk in range(4):
                        plsc.addupdate_scatter(acc, [dr1, cols[kk]],
                                               vals[4 + kk] * ws1)
                return carry2
            lax.fori_loop(0, CH // 16, blk_fn, 0)
        return carry
    lax.fori_loop(0, nbt4 >> 2, iter4, 0)

    wait_gather(0, 0)
    wait_edata(1)
    wait_edata(2)

    g0 = (c * NS + s) * BROWS
    nfw, rw = divmod(BROWS, 512)
    for q in range(nfw):
        pltpu.sync_copy(acc.at[pl.ds(q * 512, 512)],
                        out_hbm.at[pl.ds(g0 + q * 512, 512)])
    if rw:
        pltpu.sync_copy(acc.at[pl.ds(nfw * 512, rw)],
                        out_hbm.at[pl.ds(g0 + nfw * 512, rw)])


@functools.cache
def _sc_layer_fn():
  return pl.kernel(
    _layer_body,
    out_type=jax.ShapeDtypeStruct((NPAD, EMB), jnp.float32),
    mesh=plsc.VectorSubcoreMesh(core_axis_name="c", subcore_axis_name="s",
                                num_cores=NC, num_subcores=NS),
    compiler_params=pltpu.CompilerParams(use_tc_tiling_on_sc=False,
                                         needs_layout_passes=False),
    scratch_types=[
        pltpu.VMEM((BROWS, EMB), jnp.float32),
        pltpu.VMEM((4, 2, CH), jnp.int32),
        pltpu.VMEM((4, CH), jnp.float32),
        pltpu.VMEM((NS, 16), jnp.int32),
        pltpu.VMEM((2, CH, EMB), jnp.float32),
        pltpu.SemaphoreType.DMA,
        pltpu.SemaphoreType.DMA,
        pltpu.SemaphoreType.DMA,
        pltpu.SemaphoreType.DMA,
        pltpu.SemaphoreType.DMA,
        pltpu.SemaphoreType.DMA,
    ],
  )


def _mean_body(a, b, c, d, o):
    o[...] = (a[...] + b[...] + c[...] + d[...]) * 0.25


_mean = pl.pallas_call(
    _mean_body,
    grid=(250,),
    in_specs=[pl.BlockSpec((200, EMB), lambda i: (i, 0))] * 4,
    out_specs=pl.BlockSpec((200, EMB), lambda i: (i, 0)),
    out_shape=jax.ShapeDtypeStruct((N_NODES, EMB), jnp.float32),
)


def kernel(user_weight, item_weight, edge_index, edge_weight):
    x0 = jnp.concatenate(
        [user_weight, item_weight,
         jnp.zeros((NPAD - N_NODES, EMB), jnp.float32)], axis=0)
    src = edge_index[1]
    dst = edge_index[0]
    pad = EP - src.shape[0]
    src_p = jnp.concatenate([src, jnp.zeros((pad,), jnp.int32)])
    dst_p = jnp.concatenate([dst, jnp.full((pad,), N_NODES, jnp.int32)])
    w_p = jnp.concatenate([edge_weight, jnp.zeros((pad,), jnp.float32)])
    e2 = jnp.stack([src_p.reshape(EP // CH, CH),
                    dst_p.reshape(EP // CH, CH)], axis=1)
    w2 = w_p.reshape(EP // CH, CH)

    eo1, wo1, cnt1 = _partition_fn()(e2, w2)
    eo2, wo2, cnt2 = _partition2_fn()(eo1, wo1, cnt1)

    sc_layer = _sc_layer_fn()
    xs = [x0]
    cur = x0
    for _ in range(N_LAYERS):
        cur = sc_layer(cur, eo2, wo2, cnt2)
        xs.append(cur)
    return _mean(*xs)


# speedup vs baseline: 1.2513x; 1.1335x over previous
import functools

import jax
import jax.numpy as jnp
from jax import lax
from jax.experimental import pallas as pl
from jax.experimental.pallas import tpu as pltpu
from jax.experimental.pallas import tpu_sc as plsc

N_USERS = 25000
N_ITEMS = 25000
N_NODES = 50000
EMB = 64
N_LAYERS = 3

NC = 2
NS = 16
NW = NC * NS
CH = 128
EP = 802816
EPT32 = EP // NW
PNCH = EPT32 // CH
CAPB = PNCH + 4
CAPB2 = 2 * PNCH + 8

HALF0 = 25088
BROWS = HALF0 // NS
NPAD = 2 * HALF0
BMAGIC = 2675

_GATHER_DNUMS = lax.GatherDimensionNumbers(
    offset_dims=(), collapsed_slice_dims=(0,), start_index_map=(0,))


def _lane_splat(v16, e):
    idx = jnp.full((16, 1), e, jnp.int32)
    return lax.gather(v16, idx, _GATHER_DNUMS, (1,),
                      mode=lax.GatherScatterMode.PROMISE_IN_BOUNDS)


def _lane_extract(v16, lane, iota):
    return jnp.max(jnp.where(iota == lane, v16, 0))


def _part_body(e2_hbm, w2_hbm, eo_hbm, wo_hbm, cnt_hbm,
               ebuf, wbuf, cbuf, srA, dsA, wrA, srB, dsB, wrB):
    c = lax.axis_index("c")
    s = lax.axis_index("s")
    wid = c * NS + s
    iota = lax.iota(jnp.int32, 16)
    zi = jnp.zeros((16,), jnp.int32)
    zf = jnp.zeros((16,), jnp.float32)

    for g in range(16):
        srA[pl.ds(g * 16, 16)] = zi
        dsA[pl.ds(g * 16, 16)] = zi
        srB[pl.ds(g * 16, 16)] = zi
        dsB[pl.ds(g * 16, 16)] = zi
        wrA[pl.ds(g * 16, 16)] = zf
        wrB[pl.ds(g * 16, 16)] = zf

    rings = ((srA, dsA, wrA), (srB, dsB, wrB))

    def flush(h, cur):
        sr, ds_, wr = rings[h]
        rb = (cur >> 7) & 1
        blk = cur >> 7
        pltpu.sync_copy(sr.at[pl.ds(rb * 128, 128)],
                        eo_hbm.at[h, wid, blk, 0])
        pltpu.sync_copy(ds_.at[pl.ds(rb * 128, 128)],
                        eo_hbm.at[h, wid, blk, 1])
        pltpu.sync_copy(wr.at[pl.ds(rb * 128, 128)],
                        wo_hbm.at[h, wid, blk])

    def chunk(i, curs):
        curA, curB = curs
        startA, startB = curA, curB
        row = wid * PNCH + i
        pltpu.sync_copy(e2_hbm.at[row], ebuf)
        pltpu.sync_copy(w2_hbm.at[row], wbuf)
        for g in range(CH // 16):
            o = g * 16
            src16 = ebuf[0, pl.ds(o, 16)]
            dst16 = ebuf[1, pl.ds(o, 16)]
            w16 = wbuf[pl.ds(o, 16)]
            mA = dst16 < HALF0
            dloc16 = jnp.where(mA, dst16, dst16 - HALF0)
            for h, m, cur in ((0, mA, curA), (1, ~mA, curB)):
                sr, ds_, wr = rings[h]
                cm = plsc.cumsum(jnp.where(m, 1, 0))
                pos = (cur + cm - 1) & 255
                plsc.store_scatter(sr, [pos], src16, mask=m)
                plsc.store_scatter(ds_, [pos], dloc16, mask=m)
                plsc.store_scatter(wr, [pos], w16, mask=m)
                if h == 0:
                    curA = cur + jnp.max(cm)
                else:
                    curB = cur + jnp.max(cm)
        for h, st, ncur in ((0, startA, curA), (1, startB, curB)):
            @pl.when((ncur >> 7) != (st >> 7))
            def _(h=h, st=st):
                flush(h, st)
        return (curA, curB)

    curA, curB = lax.fori_loop(0, PNCH, chunk, (jnp.int32(0), jnp.int32(0)))

    nblks = []
    for h, cur in ((0, curA), (1, curB)):
        sr, ds_, wr = rings[h]
        rem = cur & 127
        for g in range(CH // 16):
            pos = (cur + g * 16 + iota) & 255
            mm = (g * 16 + iota) < (128 - rem)
            plsc.store_scatter(wr, [pos], jnp.zeros((16,), jnp.float32),
                               mask=mm)
        flush(h, cur)
        nblks.append((cur >> 7) + 1)

    cv = (jnp.where(iota == 0, nblks[0], 0)
          + jnp.where(iota == 1, nblks[1], 0))
    cbuf[pl.ds(0, 16)] = cv
    pltpu.sync_copy(cbuf, cnt_hbm.at[wid])


@functools.cache
def _partition_fn():
  return pl.kernel(
    _part_body,
    out_type=(
        jax.ShapeDtypeStruct((2, NW, CAPB, 2, CH), jnp.int32),
        jax.ShapeDtypeStruct((2, NW, CAPB, CH), jnp.float32),
        jax.ShapeDtypeStruct((NW, 16), jnp.int32),
    ),
    mesh=plsc.VectorSubcoreMesh(core_axis_name="c", subcore_axis_name="s",
                                num_cores=NC, num_subcores=NS),
    compiler_params=pltpu.CompilerParams(use_tc_tiling_on_sc=False,
                                         needs_layout_passes=False),
    scratch_types=[
        pltpu.VMEM((2, CH), jnp.int32),
        pltpu.VMEM((CH,), jnp.float32),
        pltpu.VMEM((16,), jnp.int32),
        pltpu.VMEM((256,), jnp.int32),
        pltpu.VMEM((256,), jnp.int32),
        pltpu.VMEM((256,), jnp.float32),
        pltpu.VMEM((256,), jnp.int32),
        pltpu.VMEM((256,), jnp.int32),
        pltpu.VMEM((256,), jnp.float32),
    ],
  )


def _part2_body(eo1_hbm, wo1_hbm, cnt1_hbm, eo2_hbm, wo2_hbm, cnt2_hbm,
                ebuf, wbuf, cbuf, cntv, sr2, ds2, wr2, sf, st0, st1):
    c = lax.axis_index("c")
    s = lax.axis_index("s")
    p = c * NS + s
    iota = lax.iota(jnp.int32, 16)
    zi = jnp.zeros((16,), jnp.int32)
    zf = jnp.zeros((16,), jnp.float32)

    def zring(i, carry):
        for b in range(NS):
            sr2[b, pl.ds(i * 16, 16)] = zi
            ds2[b, pl.ds(i * 16, 16)] = zi
            wr2[b, pl.ds(i * 16, 16)] = zf
        return carry
    lax.fori_loop(0, 16, zring, 0)

    t1 = s
    t2 = NS + s
    pltpu.sync_copy(cnt1_hbm.at[t1], cntv.at[0])
    pltpu.sync_copy(cnt1_hbm.at[t2], cntv.at[1])
    n1 = _lane_extract(cntv[0, pl.ds(0, 16)], c, iota)
    n2 = _lane_extract(cntv[1, pl.ds(0, 16)], c, iota)
    nbt = n1 + n2

    def fire_stage(i):
        iv = jnp.minimum(i, nbt - 1)
        t = jnp.where(iv < n1, t1, t2)
        blk = jnp.where(iv < n1, iv, iv - n1)
        b = i & 1

        @pl.when(b == 0)
        def _():
            pltpu.async_copy(eo1_hbm.at[c, t, blk], ebuf.at[0], st0)
            pltpu.async_copy(wo1_hbm.at[c, t, blk], wbuf.at[0], st0)

        @pl.when(b == 1)
        def _():
            pltpu.async_copy(eo1_hbm.at[c, t, blk], ebuf.at[1], st1)
            pltpu.async_copy(wo1_hbm.at[c, t, blk], wbuf.at[1], st1)

    def wait_stage(i):
        b = i & 1

        @pl.when(b == 0)
        def _():
            pltpu.make_async_copy(eo1_hbm.at[0, 0, 0], ebuf.at[0], st0).wait()
            pltpu.make_async_copy(wo1_hbm.at[0, 0, 0], wbuf.at[0], st0).wait()

        @pl.when(b == 1)
        def _():
            pltpu.make_async_copy(eo1_hbm.at[0, 0, 0], ebuf.at[1], st1).wait()
            pltpu.make_async_copy(wo1_hbm.at[0, 0, 0], wbuf.at[1], st1).wait()

    def flush(b, cur):
        rb = (cur >> 7) & 1
        blk = cur >> 7
        pltpu.async_copy(sr2.at[b, pl.ds(rb * 128, 128)],
                         eo2_hbm.at[p, b, blk, 0], sf)
        pltpu.async_copy(ds2.at[b, pl.ds(rb * 128, 128)],
                         eo2_hbm.at[p, b, blk, 1], sf)
        pltpu.async_copy(wr2.at[b, pl.ds(rb * 128, 128)],
                         wo2_hbm.at[p, b, blk], sf)

    def chunk(i, carry):
        curs = list(carry[:NS])
        nf = carry[NS]
        bi = i & 1
        wait_stage(i)
        fire_stage(i + 1)
        starts = list(curs)
        for g in range(CH // 16):
            o = g * 16
            src16 = ebuf[bi, 0, pl.ds(o, 16)]
            dl16 = ebuf[bi, 1, pl.ds(o, 16)]
            w16 = wbuf[bi, pl.ds(o, 16)]
            bkt16 = (dl16 * BMAGIC) >> 22
            dloc16 = dl16 - bkt16 * BROWS
            for b in range(NS):
                m = bkt16 == b
                cm = plsc.cumsum(jnp.where(m, 1, 0))
                cur = curs[b]
                pos = (cur + cm - 1) & 255
                plsc.store_scatter(sr2.at[b], [pos], src16, mask=m)
                plsc.store_scatter(ds2.at[b], [pos], dloc16, mask=m)
                plsc.store_scatter(wr2.at[b], [pos], w16, mask=m)
                curs[b] = cur + jnp.max(cm)
        for b in range(NS):
            crossed = (curs[b] >> 7) != (starts[b] >> 7)

            @pl.when(crossed)
            def _(b=b, st=starts[b]):
                flush(b, st)
            nf = nf + jnp.where(crossed, 3, 0)
        return tuple(curs) + (nf,)

    fire_stage(jnp.int32(0))
    init = tuple(jnp.int32(0) for _ in range(NS)) + (jnp.int32(0),)
    fin = lax.fori_loop(0, nbt, chunk, init)
    wait_stage(nbt)
    curs = fin[:NS]
    nf = fin[NS]

    cv = jnp.zeros((16,), jnp.int32)
    for b in range(NS):
        cur = curs[b]
        rem = cur & 127
        for g in range(CH // 16):
            pos = (cur + g * 16 + iota) & 255
            mm = (g * 16 + iota) < (128 - rem)
            plsc.store_scatter(wr2.at[b], [pos], zf, mask=mm)
        flush(b, cur)
        nf = nf + 3
        cv = cv + jnp.where(iota == b, (cur >> 7) + 1, 0)

    def drain(i, carry):
        pltpu.make_async_copy(sr2.at[0, pl.ds(0, 128)],
                              eo2_hbm.at[0, 0, 0, 0], sf).wait()
        return carry
    lax.fori_loop(0, nf, drain, 0)

    cbuf[pl.ds(0, 16)] = cv
    pltpu.sync_copy(cbuf, cnt2_hbm.at[p])


@functools.cache
def _partition2_fn():
  return pl.kernel(
    _part2_body,
    out_type=(
        jax.ShapeDtypeStruct((NW, NS, CAPB2, 2, CH), jnp.int32),
        jax.ShapeDtypeStruct((NW, NS, CAPB2, CH), jnp.float32),
        jax.ShapeDtypeStruct((NW, 16), jnp.int32),
    ),
    mesh=plsc.VectorSubcoreMesh(core_axis_name="c", subcore_axis_name="s",
                                num_cores=NC, num_subcores=NS),
    compiler_params=pltpu.CompilerParams(use_tc_tiling_on_sc=False,
                                         needs_layout_passes=False),
    scratch_types=[
        pltpu.VMEM((2, 2, CH), jnp.int32),
        pltpu.VMEM((2, CH), jnp.float32),
        pltpu.VMEM((16,), jnp.int32),
        pltpu.VMEM((2, 16), jnp.int32),
        pltpu.VMEM((NS, 256), jnp.int32),
        pltpu.VMEM((NS, 256), jnp.int32),
        pltpu.VMEM((NS, 256), jnp.float32),
        pltpu.SemaphoreType.DMA,
        pltpu.SemaphoreType.DMA,
        pltpu.SemaphoreType.DMA,
    ],
  )


def _layer_body(x_hbm, eo_hbm, wo_hbm, cnt_hbm, out_hbm,
                acc, ebuf, wbuf, cntv, rows2,
                sg0, sg1, se0, se1, se2, se3):
    c = lax.axis_index("c")
    s = lax.axis_index("s")
    sg = (sg0, sg1)
    se = (se0, se1, se2, se3)
    iota = lax.iota(jnp.int32, 16)
    cols = [(k * 16 + iota) for k in range(4)]

    for r in range(NS):
        pltpu.sync_copy(cnt_hbm.at[c * NS + r], cntv.at[r])
    cnts = jnp.zeros((16,), jnp.int32)
    for r in range(NS):
        nr = _lane_extract(cntv[r, pl.ds(0, 16)], s, iota)
        cnts = cnts + jnp.where(iota == r, nr, 0)
    prefix = plsc.cumsum(cnts)
    excl = prefix - cnts
    nbt = jnp.max(prefix)
    nbt4 = ((nbt + 3) >> 2) << 2

    zf = jnp.zeros((16,), jnp.float32)

    def zacc(i, carry):
        for k in range(4):
            acc[i, pl.ds(k * 16, 16)] = zf
        return carry
    lax.fori_loop(0, BROWS, zacc, 0)

    def fire_edata(i, q):
        iv = jnp.minimum(i, nbt - 1)
        mle = prefix <= iv
        r = jnp.max(plsc.all_reduce_population_count(mle))
        blk = iv - _lane_extract(excl, r, iota)
        pltpu.async_copy(eo_hbm.at[c * NS + r, s, blk], ebuf.at[q], se[q])
        pltpu.async_copy(wo_hbm.at[c * NS + r, s, blk], wbuf.at[q], se[q])

    def wait_edata(q):
        pltpu.make_async_copy(eo_hbm.at[0, 0, 0], ebuf.at[q], se[q]).wait()
        pltpu.make_async_copy(wo_hbm.at[0, 0, 0], wbuf.at[q], se[q]).wait()

    def fire_gather(b, q):
        pltpu.async_copy(x_hbm.at[ebuf.at[q, 0]], rows2.at[b], sg[b])

    def wait_gather(b, q):
        pltpu.make_async_copy(x_hbm.at[ebuf.at[q, 0]], rows2.at[b],
                              sg[b]).wait()

    fire_edata(jnp.int32(0), 0)
    fire_edata(jnp.int32(1), 1)
    wait_edata(0)
    fire_gather(0, 0)
    fire_edata(jnp.int32(2), 2)

    def iter4(i4, carry):
        for k in range(4):
            ci = i4 * 4 + k
            q = k
            b = k & 1
            wait_edata((k + 1) % 4)
            fire_gather(1 - b, (k + 1) % 4)
            fire_edata(ci + 3, (k + 3) % 4)
            wait_gather(b, q)
            valid = ci < nbt

            def blk_fn(bi, carry2, _b=b, _q=q, _valid=valid):
                jb = bi * 16
                w16 = jnp.where(_valid, wbuf[_q, pl.ds(jb, 16)], 0.0)
                dl16 = ebuf[_q, 1, pl.ds(jb, 16)]
                for e in range(0, 16, 2):
                    j0 = jb + e
                    j1 = jb + e + 1
                    ws0 = _lane_splat(w16, e)
                    ws1 = _lane_splat(w16, e + 1)
                    dr0 = _lane_splat(dl16, e)
                    dr1 = _lane_splat(dl16, e + 1)
                    vals = [rows2[_b, j0, pl.ds(kk * 16, 16)]
                            for kk in range(4)]
                    vals += [rows2[_b, j1, pl.ds(kk * 16, 16)]
                             for kk in range(4)]
                    for kk in range(4):
                        plsc.addupdate_scatter(acc, [dr0, cols[kk]],
                                               vals[kk] * ws0)
                    for kk in range(4):
                        plsc.addupdate_scatter(acc, [dr1, cols[kk]],
                                               vals[4 + kk] * ws1)
                return carry2
            lax.fori_loop(0, CH // 16, blk_fn, 0)
        return carry
    lax.fori_loop(0, nbt4 >> 2, iter4, 0)

    wait_gather(0, 0)
    wait_edata(1)
    wait_edata(2)

    g0 = (c * NS + s) * BROWS
    nfw, rw = divmod(BROWS, 512)
    for q in range(nfw):
        pltpu.sync_copy(acc.at[pl.ds(q * 512, 512)],
                        out_hbm.at[pl.ds(g0 + q * 512, 512)])
    if rw:
        pltpu.sync_copy(acc.at[pl.ds(nfw * 512, rw)],
                        out_hbm.at[pl.ds(g0 + nfw * 512, rw)])


@functools.cache
def _sc_layer_fn():
  return pl.kernel(
    _layer_body,
    out_type=jax.ShapeDtypeStruct((NPAD, EMB), jnp.float32),
    mesh=plsc.VectorSubcoreMesh(core_axis_name="c", subcore_axis_name="s",
                                num_cores=NC, num_subcores=NS),
    compiler_params=pltpu.CompilerParams(use_tc_tiling_on_sc=False,
                                         needs_layout_passes=False),
    scratch_types=[
        pltpu.VMEM((BROWS, EMB), jnp.float32),
        pltpu.VMEM((4, 2, CH), jnp.int32),
        pltpu.VMEM((4, CH), jnp.float32),
        pltpu.VMEM((NS, 16), jnp.int32),
        pltpu.VMEM((2, CH, EMB), jnp.float32),
        pltpu.SemaphoreType.DMA,
        pltpu.SemaphoreType.DMA,
        pltpu.SemaphoreType.DMA,
        pltpu.SemaphoreType.DMA,
        pltpu.SemaphoreType.DMA,
        pltpu.SemaphoreType.DMA,
    ],
  )


def _mean_body(a, b, c, d, o):
    o[...] = (a[...] + b[...] + c[...] + d[...]) * 0.25


_mean = pl.pallas_call(
    _mean_body,
    grid=(250,),
    in_specs=[pl.BlockSpec((200, EMB), lambda i: (i, 0))] * 4,
    out_specs=pl.BlockSpec((200, EMB), lambda i: (i, 0)),
    out_shape=jax.ShapeDtypeStruct((N_NODES, EMB), jnp.float32),
)


def kernel(user_weight, item_weight, edge_index, edge_weight):
    x0 = jnp.concatenate(
        [user_weight, item_weight,
         jnp.zeros((NPAD - N_NODES, EMB), jnp.float32)], axis=0)
    src = edge_index[1]
    dst = edge_index[0]
    pad = EP - src.shape[0]
    src_p = jnp.concatenate([src, jnp.zeros((pad,), jnp.int32)])
    dst_p = jnp.concatenate([dst, jnp.full((pad,), N_NODES, jnp.int32)])
    w_p = jnp.concatenate([edge_weight, jnp.zeros((pad,), jnp.float32)])
    e2 = jnp.stack([src_p.reshape(EP // CH, CH),
                    dst_p.reshape(EP // CH, CH)], axis=1)
    w2 = w_p.reshape(EP // CH, CH)

    eo1, wo1, cnt1 = _partition_fn()(e2, w2)
    eo2, wo2, cnt2 = _partition2_fn()(eo1, wo1, cnt1)

    sc_layer = _sc_layer_fn()
    xs = [x0]
    cur = x0
    for _ in range(N_LAYERS):
        cur = sc_layer(cur, eo2, wo2, cnt2)
        xs.append(cur)
    return _mean(*xs)


# speedup vs baseline: 1.3903x; 1.1111x over previous
import functools

import jax
import jax.numpy as jnp
from jax import lax
from jax.experimental import pallas as pl
from jax.experimental.pallas import tpu as pltpu
from jax.experimental.pallas import tpu_sc as plsc

N_USERS = 25000
N_ITEMS = 25000
N_NODES = 50000
EMB = 64
N_LAYERS = 3

NC = 2
NS = 16
NW = NC * NS
CH = 128
EP = 802816
EPT32 = EP // NW
PNCH = EPT32 // CH
CAPB = PNCH + 4
CAPB2 = 2 * PNCH + 8

HALF0 = 25088
BROWS = HALF0 // NS
NPAD = 2 * HALF0
BMAGIC = 2675

_GATHER_DNUMS = lax.GatherDimensionNumbers(
    offset_dims=(), collapsed_slice_dims=(0,), start_index_map=(0,))


def _lane_splat(v16, e):
    idx = jnp.full((16, 1), e, jnp.int32)
    return lax.gather(v16, idx, _GATHER_DNUMS, (1,),
                      mode=lax.GatherScatterMode.PROMISE_IN_BOUNDS)


def _lane_extract(v16, lane, iota):
    return jnp.max(jnp.where(iota == lane, v16, 0))


def _part_body(e2_hbm, w2_hbm, eo_hbm, wo_hbm, cnt_hbm,
               ebuf, wbuf, cbuf, srA, dsA, wrA, srB, dsB, wrB, st0, st1):
    c = lax.axis_index("c")
    s = lax.axis_index("s")
    wid = c * NS + s
    iota = lax.iota(jnp.int32, 16)
    zi = jnp.zeros((16,), jnp.int32)
    zf = jnp.zeros((16,), jnp.float32)

    for g in range(16):
        srA[pl.ds(g * 16, 16)] = zi
        dsA[pl.ds(g * 16, 16)] = zi
        srB[pl.ds(g * 16, 16)] = zi
        dsB[pl.ds(g * 16, 16)] = zi
        wrA[pl.ds(g * 16, 16)] = zf
        wrB[pl.ds(g * 16, 16)] = zf

    rings = ((srA, dsA, wrA), (srB, dsB, wrB))

    def fire_stage(i):
        row = wid * PNCH + jnp.minimum(i, PNCH - 1)
        b = i & 1

        @pl.when(b == 0)
        def _():
            pltpu.async_copy(e2_hbm.at[row], ebuf.at[0], st0)
            pltpu.async_copy(w2_hbm.at[row], wbuf.at[0], st0)

        @pl.when(b == 1)
        def _():
            pltpu.async_copy(e2_hbm.at[row], ebuf.at[1], st1)
            pltpu.async_copy(w2_hbm.at[row], wbuf.at[1], st1)

    def wait_stage(i):
        b = i & 1

        @pl.when(b == 0)
        def _():
            pltpu.make_async_copy(e2_hbm.at[0], ebuf.at[0], st0).wait()
            pltpu.make_async_copy(w2_hbm.at[0], wbuf.at[0], st0).wait()

        @pl.when(b == 1)
        def _():
            pltpu.make_async_copy(e2_hbm.at[0], ebuf.at[1], st1).wait()
            pltpu.make_async_copy(w2_hbm.at[0], wbuf.at[1], st1).wait()

    def flush(h, cur):
        sr, ds_, wr = rings[h]
        rb = (cur >> 7) & 1
        blk = cur >> 7
        pltpu.sync_copy(sr.at[pl.ds(rb * 128, 128)],
                        eo_hbm.at[h, wid, blk, 0])
        pltpu.sync_copy(ds_.at[pl.ds(rb * 128, 128)],
                        eo_hbm.at[h, wid, blk, 1])
        pltpu.sync_copy(wr.at[pl.ds(rb * 128, 128)],
                        wo_hbm.at[h, wid, blk])

    def chunk(i, curs):
        curA, curB = curs
        startA, startB = curA, curB
        bi = i & 1
        wait_stage(i)
        fire_stage(i + 1)
        for g in range(CH // 16):
            o = g * 16
            src16 = ebuf[bi, 0, pl.ds(o, 16)]
            dst16 = ebuf[bi, 1, pl.ds(o, 16)]
            w16 = wbuf[bi, pl.ds(o, 16)]
            mA = dst16 < HALF0
            dloc16 = jnp.where(mA, dst16, dst16 - HALF0)
            for h, m, cur in ((0, mA, curA), (1, ~mA, curB)):
                sr, ds_, wr = rings[h]
                cm = plsc.cumsum(jnp.where(m, 1, 0))
                pos = (cur + cm - 1) & 255
                plsc.store_scatter(sr, [pos], src16, mask=m)
                plsc.store_scatter(ds_, [pos], dloc16, mask=m)
                plsc.store_scatter(wr, [pos], w16, mask=m)
                if h == 0:
                    curA = cur + jnp.max(cm)
                else:
                    curB = cur + jnp.max(cm)
        for h, st, ncur in ((0, startA, curA), (1, startB, curB)):
            @pl.when((ncur >> 7) != (st >> 7))
            def _(h=h, st=st):
                flush(h, st)
        return (curA, curB)

    fire_stage(jnp.int32(0))
    curA, curB = lax.fori_loop(0, PNCH, chunk, (jnp.int32(0), jnp.int32(0)))
    wait_stage(jnp.int32(PNCH))

    nblks = []
    for h, cur in ((0, curA), (1, curB)):
        sr, ds_, wr = rings[h]
        rem = cur & 127
        for g in range(CH // 16):
            pos = (cur + g * 16 + iota) & 255
            mm = (g * 16 + iota) < (128 - rem)
            plsc.store_scatter(wr, [pos], jnp.zeros((16,), jnp.float32),
                               mask=mm)
        flush(h, cur)
        nblks.append((cur >> 7) + 1)

    cv = (jnp.where(iota == 0, nblks[0], 0)
          + jnp.where(iota == 1, nblks[1], 0))
    cbuf[pl.ds(0, 16)] = cv
    pltpu.sync_copy(cbuf, cnt_hbm.at[wid])


@functools.cache
def _partition_fn():
  return pl.kernel(
    _part_body,
    out_type=(
        jax.ShapeDtypeStruct((2, NW, CAPB, 2, CH), jnp.int32),
        jax.ShapeDtypeStruct((2, NW, CAPB, CH), jnp.float32),
        jax.ShapeDtypeStruct((NW, 16), jnp.int32),
    ),
    mesh=plsc.VectorSubcoreMesh(core_axis_name="c", subcore_axis_name="s",
                                num_cores=NC, num_subcores=NS),
    compiler_params=pltpu.CompilerParams(use_tc_tiling_on_sc=False,
                                         needs_layout_passes=False),
    scratch_types=[
        pltpu.VMEM((2, 2, CH), jnp.int32),
        pltpu.VMEM((2, CH), jnp.float32),
        pltpu.VMEM((16,), jnp.int32),
        pltpu.VMEM((256,), jnp.int32),
        pltpu.VMEM((256,), jnp.int32),
        pltpu.VMEM((256,), jnp.float32),
        pltpu.VMEM((256,), jnp.int32),
        pltpu.VMEM((256,), jnp.int32),
        pltpu.VMEM((256,), jnp.float32),
        pltpu.SemaphoreType.DMA,
        pltpu.SemaphoreType.DMA,
    ],
  )


def _part2_body(eo1_hbm, wo1_hbm, cnt1_hbm, eo2_hbm, wo2_hbm, cnt2_hbm,
                ebuf, wbuf, cbuf, cntv, sr2, ds2, wr2, sf, st0, st1):
    c = lax.axis_index("c")
    s = lax.axis_index("s")
    p = c * NS + s
    iota = lax.iota(jnp.int32, 16)
    zi = jnp.zeros((16,), jnp.int32)
    zf = jnp.zeros((16,), jnp.float32)

    def zring(i, carry):
        for b in range(NS):
            sr2[b, pl.ds(i * 16, 16)] = zi
            ds2[b, pl.ds(i * 16, 16)] = zi
            wr2[b, pl.ds(i * 16, 16)] = zf
        return carry
    lax.fori_loop(0, 16, zring, 0)

    t1 = s
    t2 = NS + s
    pltpu.sync_copy(cnt1_hbm.at[t1], cntv.at[0])
    pltpu.sync_copy(cnt1_hbm.at[t2], cntv.at[1])
    n1 = _lane_extract(cntv[0, pl.ds(0, 16)], c, iota)
    n2 = _lane_extract(cntv[1, pl.ds(0, 16)], c, iota)
    nbt = n1 + n2

    def fire_stage(i):
        iv = jnp.minimum(i, nbt - 1)
        t = jnp.where(iv < n1, t1, t2)
        blk = jnp.where(iv < n1, iv, iv - n1)
        b = i & 1

        @pl.when(b == 0)
        def _():
            pltpu.async_copy(eo1_hbm.at[c, t, blk], ebuf.at[0], st0)
            pltpu.async_copy(wo1_hbm.at[c, t, blk], wbuf.at[0], st0)

        @pl.when(b == 1)
        def _():
            pltpu.async_copy(eo1_hbm.at[c, t, blk], ebuf.at[1], st1)
            pltpu.async_copy(wo1_hbm.at[c, t, blk], wbuf.at[1], st1)

    def wait_stage(i):
        b = i & 1

        @pl.when(b == 0)
        def _():
            pltpu.make_async_copy(eo1_hbm.at[0, 0, 0], ebuf.at[0], st0).wait()
            pltpu.make_async_copy(wo1_hbm.at[0, 0, 0], wbuf.at[0], st0).wait()

        @pl.when(b == 1)
        def _():
            pltpu.make_async_copy(eo1_hbm.at[0, 0, 0], ebuf.at[1], st1).wait()
            pltpu.make_async_copy(wo1_hbm.at[0, 0, 0], wbuf.at[1], st1).wait()

    def flush(b, cur):
        rb = (cur >> 7) & 1
        blk = cur >> 7
        pltpu.async_copy(sr2.at[b, pl.ds(rb * 128, 128)],
                         eo2_hbm.at[p, b, blk, 0], sf)
        pltpu.async_copy(ds2.at[b, pl.ds(rb * 128, 128)],
                         eo2_hbm.at[p, b, blk, 1], sf)
        pltpu.async_copy(wr2.at[b, pl.ds(rb * 128, 128)],
                         wo2_hbm.at[p, b, blk], sf)

    def chunk(i, carry):
        curs = list(carry[:NS])
        nf = carry[NS]
        bi = i & 1
        wait_stage(i)
        fire_stage(i + 1)
        starts = list(curs)
        for g in range(CH // 16):
            o = g * 16
            src16 = ebuf[bi, 0, pl.ds(o, 16)]
            dl16 = ebuf[bi, 1, pl.ds(o, 16)]
            w16 = wbuf[bi, pl.ds(o, 16)]
            bkt16 = (dl16 * BMAGIC) >> 22
            dloc16 = dl16 - bkt16 * BROWS
            for b in range(NS):
                m = bkt16 == b
                cm = plsc.cumsum(jnp.where(m, 1, 0))
                cur = curs[b]
                pos = (cur + cm - 1) & 255
                plsc.store_scatter(sr2.at[b], [pos], src16, mask=m)
                plsc.store_scatter(ds2.at[b], [pos], dloc16, mask=m)
                plsc.store_scatter(wr2.at[b], [pos], w16, mask=m)
                curs[b] = cur + jnp.max(cm)
        for b in range(NS):
            crossed = (curs[b] >> 7) != (starts[b] >> 7)

            @pl.when(crossed)
            def _(b=b, st=starts[b]):
                flush(b, st)
            nf = nf + jnp.where(crossed, 3, 0)
        return tuple(curs) + (nf,)

    fire_stage(jnp.int32(0))
    init = tuple(jnp.int32(0) for _ in range(NS)) + (jnp.int32(0),)
    fin = lax.fori_loop(0, nbt, chunk, init)
    wait_stage(nbt)
    curs = fin[:NS]
    nf = fin[NS]

    cv = jnp.zeros((16,), jnp.int32)
    for b in range(NS):
        cur = curs[b]
        rem = cur & 127
        for g in range(CH // 16):
            pos = (cur + g * 16 + iota) & 255
            mm = (g * 16 + iota) < (128 - rem)
            plsc.store_scatter(wr2.at[b], [pos], zf, mask=mm)
        flush(b, cur)
        nf = nf + 3
        cv = cv + jnp.where(iota == b, (cur >> 7) + 1, 0)

    def drain(i, carry):
        pltpu.make_async_copy(sr2.at[0, pl.ds(0, 128)],
                              eo2_hbm.at[0, 0, 0, 0], sf).wait()
        return carry
    lax.fori_loop(0, nf, drain, 0)

    cbuf[pl.ds(0, 16)] = cv
    pltpu.sync_copy(cbuf, cnt2_hbm.at[p])


@functools.cache
def _partition2_fn():
  return pl.kernel(
    _part2_body,
    out_type=(
        jax.ShapeDtypeStruct((NW, NS, CAPB2, 2, CH), jnp.int32),
        jax.ShapeDtypeStruct((NW, NS, CAPB2, CH), jnp.float32),
        jax.ShapeDtypeStruct((NW, 16), jnp.int32),
    ),
    mesh=plsc.VectorSubcoreMesh(core_axis_name="c", subcore_axis_name="s",
                                num_cores=NC, num_subcores=NS),
    compiler_params=pltpu.CompilerParams(use_tc_tiling_on_sc=False,
                                         needs_layout_passes=False),
    scratch_types=[
        pltpu.VMEM((2, 2, CH), jnp.int32),
        pltpu.VMEM((2, CH), jnp.float32),
        pltpu.VMEM((16,), jnp.int32),
        pltpu.VMEM((2, 16), jnp.int32),
        pltpu.VMEM((NS, 256), jnp.int32),
        pltpu.VMEM((NS, 256), jnp.int32),
        pltpu.VMEM((NS, 256), jnp.float32),
        pltpu.SemaphoreType.DMA,
        pltpu.SemaphoreType.DMA,
        pltpu.SemaphoreType.DMA,
    ],
  )


def _layer_body(x_hbm, eo_hbm, wo_hbm, cnt_hbm, out_hbm,
                acc, ebuf, wbuf, cntv, rows2,
                sg0, sg1, se0, se1, se2, se3):
    c = lax.axis_index("c")
    s = lax.axis_index("s")
    sg = (sg0, sg1)
    se = (se0, se1, se2, se3)
    iota = lax.iota(jnp.int32, 16)
    cols = [(k * 16 + iota) for k in range(4)]

    for r in range(NS):
        pltpu.sync_copy(cnt_hbm.at[c * NS + r], cntv.at[r])
    cnts = jnp.zeros((16,), jnp.int32)
    for r in range(NS):
        nr = _lane_extract(cntv[r, pl.ds(0, 16)], s, iota)
        cnts = cnts + jnp.where(iota == r, nr, 0)
    prefix = plsc.cumsum(cnts)
    excl = prefix - cnts
    nbt = jnp.max(prefix)
    nbt4 = ((nbt + 3) >> 2) << 2

    zf = jnp.zeros((16,), jnp.float32)

    def zacc(i, carry):
        for k in range(4):
            acc[i, pl.ds(k * 16, 16)] = zf
        return carry
    lax.fori_loop(0, BROWS, zacc, 0)

    def fire_edata(i, q):
        iv = jnp.minimum(i, nbt - 1)
        mle = prefix <= iv
        r = jnp.max(plsc.all_reduce_population_count(mle))
        blk = iv - _lane_extract(excl, r, iota)
        pltpu.async_copy(eo_hbm.at[c * NS + r, s, blk], ebuf.at[q], se[q])
        pltpu.async_copy(wo_hbm.at[c * NS + r, s, blk], wbuf.at[q], se[q])

    def wait_edata(q):
        pltpu.make_async_copy(eo_hbm.at[0, 0, 0], ebuf.at[q], se[q]).wait()
        pltpu.make_async_copy(wo_hbm.at[0, 0, 0], wbuf.at[q], se[q]).wait()

    def fire_gather(b, q):
        pltpu.async_copy(x_hbm.at[ebuf.at[q, 0]], rows2.at[b], sg[b])

    def wait_gather(b, q):
        pltpu.make_async_copy(x_hbm.at[ebuf.at[q, 0]], rows2.at[b],
                              sg[b]).wait()

    fire_edata(jnp.int32(0), 0)
    fire_edata(jnp.int32(1), 1)
    wait_edata(0)
    fire_gather(0, 0)
    fire_edata(jnp.int32(2), 2)

    def iter4(i4, carry):
        for k in range(4):
            ci = i4 * 4 + k
            q = k
            b = k & 1
            wait_edata((k + 1) % 4)
            fire_gather(1 - b, (k + 1) % 4)
            fire_edata(ci + 3, (k + 3) % 4)
            wait_gather(b, q)
            valid = ci < nbt

            def blk_fn(bi, carry2, _b=b, _q=q, _valid=valid):
                jb = bi * 16
                w16 = jnp.where(_valid, wbuf[_q, pl.ds(jb, 16)], 0.0)
                dl16 = ebuf[_q, 1, pl.ds(jb, 16)]
                for e in range(0, 16, 2):
                    j0 = jb + e
                    j1 = jb + e + 1
                    ws0 = _lane_splat(w16, e)
                    ws1 = _lane_splat(w16, e + 1)
                    dr0 = _lane_splat(dl16, e)
                    dr1 = _lane_splat(dl16, e + 1)
                    vals = [rows2[_b, j0, pl.ds(kk * 16, 16)]
                            for kk in range(4)]
                    vals += [rows2[_b, j1, pl.ds(kk * 16, 16)]
                             for kk in range(4)]
                    for kk in range(4):
                        plsc.addupdate_scatter(acc, [dr0, cols[kk]],
                                               vals[kk] * ws0)
                    for kk in range(4):
                        plsc.addupdate_scatter(acc, [dr1, cols[kk]],
                                               vals[4 + kk] * ws1)
                return carry2
            lax.fori_loop(0, CH // 16, blk_fn, 0)
        return carry
    lax.fori_loop(0, nbt4 >> 2, iter4, 0)

    wait_gather(0, 0)
    wait_edata(1)
    wait_edata(2)

    g0 = (c * NS + s) * BROWS
    nfw, rw = divmod(BROWS, 512)
    for q in range(nfw):
        pltpu.sync_copy(acc.at[pl.ds(q * 512, 512)],
                        out_hbm.at[pl.ds(g0 + q * 512, 512)])
    if rw:
        pltpu.sync_copy(acc.at[pl.ds(nfw * 512, rw)],
                        out_hbm.at[pl.ds(g0 + nfw * 512, rw)])


@functools.cache
def _sc_layer_fn():
  return pl.kernel(
    _layer_body,
    out_type=jax.ShapeDtypeStruct((NPAD, EMB), jnp.float32),
    mesh=plsc.VectorSubcoreMesh(core_axis_name="c", subcore_axis_name="s",
                                num_cores=NC, num_subcores=NS),
    compiler_params=pltpu.CompilerParams(use_tc_tiling_on_sc=False,
                                         needs_layout_passes=False),
    scratch_types=[
        pltpu.VMEM((BROWS, EMB), jnp.float32),
        pltpu.VMEM((4, 2, CH), jnp.int32),
        pltpu.VMEM((4, CH), jnp.float32),
        pltpu.VMEM((NS, 16), jnp.int32),
        pltpu.VMEM((2, CH, EMB), jnp.float32),
        pltpu.SemaphoreType.DMA,
        pltpu.SemaphoreType.DMA,
        pltpu.SemaphoreType.DMA,
        pltpu.SemaphoreType.DMA,
        pltpu.SemaphoreType.DMA,
        pltpu.SemaphoreType.DMA,
    ],
  )


def _mean_body(a, b, c, d, o):
    o[...] = (a[...] + b[...] + c[...] + d[...]) * 0.25


_mean = pl.pallas_call(
    _mean_body,
    grid=(250,),
    in_specs=[pl.BlockSpec((200, EMB), lambda i: (i, 0))] * 4,
    out_specs=pl.BlockSpec((200, EMB), lambda i: (i, 0)),
    out_shape=jax.ShapeDtypeStruct((N_NODES, EMB), jnp.float32),
)


def kernel(user_weight, item_weight, edge_index, edge_weight):
    x0 = jnp.concatenate(
        [user_weight, item_weight,
         jnp.zeros((NPAD - N_NODES, EMB), jnp.float32)], axis=0)
    src = edge_index[1]
    dst = edge_index[0]
    pad = EP - src.shape[0]
    src_p = jnp.concatenate([src, jnp.zeros((pad,), jnp.int32)])
    dst_p = jnp.concatenate([dst, jnp.full((pad,), N_NODES, jnp.int32)])
    w_p = jnp.concatenate([edge_weight, jnp.zeros((pad,), jnp.float32)])
    e2 = jnp.stack([src_p.reshape(EP // CH, CH),
                    dst_p.reshape(EP // CH, CH)], axis=1)
    w2 = w_p.reshape(EP // CH, CH)

    eo1, wo1, cnt1 = _partition_fn()(e2, w2)
    eo2, wo2, cnt2 = _partition2_fn()(eo1, wo1, cnt1)

    sc_layer = _sc_layer_fn()
    xs = [x0]
    cur = x0
    for _ in range(N_LAYERS):
        cur = sc_layer(cur, eo2, wo2, cnt2)
        xs.append(cur)
    return _mean(*xs)
